# Initial kernel scaffold; baseline (speedup 1.0000x reference)
import jax
import jax.numpy as jnp
from jax.experimental import pallas as pl


def kernel(x_node, edge_index_node_to_node, W1, a_src1, a_dst1, b1, W2, a_src2, a_dst2, b2, lin_W, lin_b):
    raise NotImplementedError("write your pallas kernel here")


# speedup vs baseline: 114.1782x; 114.1782x over previous
import functools

import jax
import jax.numpy as jnp
from jax import lax
from jax.experimental import pallas as pl
from jax.experimental.pallas import tpu as pltpu
from jax.experimental.pallas import tpu_sc as plsc

N = 1024
E = 32768
H = 2

NS = 16
EPW = E // NS
CHUNK = 128
NCH = EPW // CHUNK
SLICE = (N * N) // NS
ZBUF = 8192


def _pre_body(o1, x_ref, w_ref, asrc_ref, adst_ref, xp_ref, asn_ref, adn_ref):
    xp = jnp.dot(x_ref[...], w_ref[...], preferred_element_type=jnp.float32)
    xp_ref[...] = xp
    ts = xp * asrc_ref[...]
    td = xp * adst_ref[...]
    asn_ref[...] = jnp.concatenate(
        [jnp.sum(ts[:, h * o1:(h + 1) * o1], axis=1, keepdims=True) for h in range(H)],
        axis=1)
    adn_ref[...] = jnp.concatenate(
        [jnp.sum(td[:, h * o1:(h + 1) * o1], axis=1, keepdims=True) for h in range(H)],
        axis=1)


def _pre(x, W, a_src_flat, a_dst_flat):
    dout = W.shape[1]
    o1 = dout // H
    return pl.pallas_call(
        functools.partial(_pre_body, o1),
        out_shape=[
            jax.ShapeDtypeStruct((N, dout), jnp.float32),
            jax.ShapeDtypeStruct((N, H), jnp.float32),
            jax.ShapeDtypeStruct((N, H), jnp.float32),
        ],
    )(x, W, a_src_flat, a_dst_flat)


def _abuild_body(src_hbm, dst_hbm, asn_hbm, adn_hbm, out_hbm,
                 src_v, dst_v, asn_v, adn_v, vals_v, idx_v, zero_v, a_sh, sem):
    c = lax.axis_index("c")
    s = lax.axis_index("s")

    ebase = s * EPW
    pltpu.sync_copy(src_hbm.at[pl.ds(ebase, EPW)], src_v)
    pltpu.sync_copy(dst_hbm.at[pl.ds(ebase, EPW)], dst_v)
    pltpu.sync_copy(asn_hbm, asn_v)
    pltpu.sync_copy(adn_hbm, adn_v)

    def zbody(i, carry):
        zero_v[pl.ds(i * 16, 16)] = jnp.zeros((16,), jnp.float32)
        return carry
    lax.fori_loop(0, ZBUF // 16, zbody, 0)
    abase = s * SLICE
    for k in range(SLICE // ZBUF):
        pltpu.sync_copy(zero_v, a_sh.at[pl.ds(abase + k * ZBUF, ZBUF)])

    for r in range(NCH):
        for q in range(CHUNK // 16):
            off = r * CHUNK + q * 16
            s16 = src_v[pl.ds(off, 16)]
            d16 = dst_v[pl.ds(off, 16)]
            av = plsc.load_gather(asn_v, [s16 * H + c])
            bv = plsc.load_gather(adn_v, [d16 * H + c])
            e = av + bv
            e = jnp.where(e >= 0.0, e, e * 0.2)
            vals_v[r, pl.ds(q * 16, 16)] = jnp.exp(e)
            idx_v[r, pl.ds(q * 16, 16)] = d16 * N + s16

    plsc.subcore_barrier()

    for r in range(NCH):
        pltpu.sync_copy(vals_v.at[r], a_sh.at[idx_v.at[r]], add=True)

    plsc.subcore_barrier()
    pltpu.async_copy(a_sh.at[pl.ds(abase, SLICE)], out_hbm.at[c, s], sem).wait()


def _abuild(src, dst, asn_flat, adn_flat):
    mesh = plsc.VectorSubcoreMesh(core_axis_name="c", subcore_axis_name="s")
    k = pl.kernel(
        _abuild_body,
        out_type=jax.ShapeDtypeStruct((H, NS, SLICE), jnp.float32),
        mesh=mesh,
        compiler_params=pltpu.CompilerParams(needs_layout_passes=False),
        scratch_types=[
            pltpu.VMEM((EPW,), jnp.int32),
            pltpu.VMEM((EPW,), jnp.int32),
            pltpu.VMEM((N * H,), jnp.float32),
            pltpu.VMEM((N * H,), jnp.float32),
            pltpu.VMEM((NCH, CHUNK), jnp.float32),
            pltpu.VMEM((NCH, CHUNK), jnp.int32),
            pltpu.VMEM((ZBUF,), jnp.float32),
            pltpu.VMEM_SHARED((N * N,), jnp.float32),
            pltpu.SemaphoreType.DMA,
        ],
    )
    return k(src, dst, asn_flat, adn_flat)


def _agg_body(relu, o1, a_ref, xp_ref, b_ref, out_ref):
    cols = []
    for h in range(H):
        A = a_ref[h]
        U = jnp.dot(A, xp_ref[:, h * o1:(h + 1) * o1],
                    preferred_element_type=jnp.float32)
        den = jnp.sum(A, axis=1, keepdims=True)
        cols.append(U / (den + 1e-16))
    out = jnp.concatenate(cols, axis=1) + b_ref[...]
    if relu:
        out = jnp.maximum(out, 0.0)
    out_ref[...] = out


def _agg(A, xp, b_row, relu):
    dout = xp.shape[1]
    o1 = dout // H
    return pl.pallas_call(
        functools.partial(_agg_body, relu, o1),
        out_shape=jax.ShapeDtypeStruct((N, dout), jnp.float32),
    )(A, xp, b_row)


def _pair_body(h_ref, w_ref, b_ref, out_ref):
    sv = jnp.dot(h_ref[...], w_ref[...], preferred_element_type=jnp.float32)
    st = lax.dot_general(w_ref[...], h_ref[...], (((0,), (1,)), ((), ())),
                         preferred_element_type=jnp.float32)
    out_ref[...] = sv + st + b_ref[0, 0]


def _pair(h, lin_W, lin_b_2d):
    return pl.pallas_call(
        _pair_body,
        out_shape=jax.ShapeDtypeStruct((N, N), jnp.float32),
    )(h, lin_W, lin_b_2d)


def _gat_layer(x, edge_src, edge_dst, W, a_src, a_dst, b, relu):
    xp, asn, adn = _pre(x, W, a_src.reshape(1, -1), a_dst.reshape(1, -1))
    A = _abuild(edge_src, edge_dst, asn.reshape(-1), adn.reshape(-1))
    A = A.reshape(H, N, N)
    return _agg(A, xp, b.reshape(1, -1), relu)


@jax.jit
def kernel(x_node, edge_index_node_to_node, W1, a_src1, a_dst1, b1,
           W2, a_src2, a_dst2, b2, lin_W, lin_b):
    src = edge_index_node_to_node[0].astype(jnp.int32)
    dst = edge_index_node_to_node[1].astype(jnp.int32)
    h1 = _gat_layer(x_node, src, dst, W1, a_src1, a_dst1, b1, relu=True)
    h2 = _gat_layer(h1, src, dst, W2, a_src2, a_dst2, b2, relu=False)
    out = _pair(h2, lin_W, lin_b.reshape(1, 1))
    return out.reshape(N * N, 1)


# speedup vs baseline: 134.6103x; 1.1789x over previous
import functools

import jax
import jax.numpy as jnp
from jax import lax
from jax.experimental import pallas as pl
from jax.experimental.pallas import tpu as pltpu
from jax.experimental.pallas import tpu_sc as plsc

N = 1024
E = 32768
H = 2

NS = 16
EPW = E // NS
CHUNK = 128
NCH = EPW // CHUNK
SLICE = (N * N) // NS
ZBUF = 8192


def _attn_logits(xp, a_flat_ref, o1):
    t = xp * a_flat_ref[...]
    return jnp.concatenate(
        [jnp.sum(t[:, h * o1:(h + 1) * o1], axis=1, keepdims=True) for h in range(H)],
        axis=1)


def _normed_agg(a_ref, xp, o1):
    cols = []
    for h in range(H):
        A = a_ref[h]
        U = jnp.dot(A, xp[:, h * o1:(h + 1) * o1],
                    preferred_element_type=jnp.float32)
        den = jnp.sum(A, axis=1, keepdims=True)
        cols.append(U / (den + 1e-16))
    return jnp.concatenate(cols, axis=1)


def _pre_body(o1, x_ref, w_ref, asrc_ref, adst_ref, xp_ref, asn_ref, adn_ref):
    xp = jnp.dot(x_ref[...], w_ref[...], preferred_element_type=jnp.float32)
    xp_ref[...] = xp
    asn_ref[...] = _attn_logits(xp, asrc_ref, o1)
    adn_ref[...] = _attn_logits(xp, adst_ref, o1)


def _pre(x, W, a_src_flat, a_dst_flat):
    dout = W.shape[1]
    return pl.pallas_call(
        functools.partial(_pre_body, dout // H),
        out_shape=[
            jax.ShapeDtypeStruct((N, dout), jnp.float32),
            jax.ShapeDtypeStruct((N, H), jnp.float32),
            jax.ShapeDtypeStruct((N, H), jnp.float32),
        ],
    )(x, W, a_src_flat, a_dst_flat)


def _mid_body(o1, o2, a_ref, xp_ref, b_ref, w2_ref, asrc_ref, adst_ref,
              xp2_ref, asn_ref, adn_ref):
    h1 = jnp.maximum(_normed_agg(a_ref, xp_ref[...], o1) + b_ref[...], 0.0)
    xp2 = jnp.dot(h1, w2_ref[...], preferred_element_type=jnp.float32)
    xp2_ref[...] = xp2
    asn_ref[...] = _attn_logits(xp2, asrc_ref, o2)
    adn_ref[...] = _attn_logits(xp2, adst_ref, o2)


def _mid(A1, xp1, b1_row, W2, a_src_flat, a_dst_flat):
    o1 = xp1.shape[1] // H
    dout = W2.shape[1]
    return pl.pallas_call(
        functools.partial(_mid_body, o1, dout // H),
        out_shape=[
            jax.ShapeDtypeStruct((N, dout), jnp.float32),
            jax.ShapeDtypeStruct((N, H), jnp.float32),
            jax.ShapeDtypeStruct((N, H), jnp.float32),
        ],
    )(A1, xp1, b1_row, W2, a_src_flat, a_dst_flat)


def _fin_body(o2, a_ref, xp_ref, b_ref, w_ref, lb_ref, out_ref):
    h2 = _normed_agg(a_ref, xp_ref[...], o2) + b_ref[...]
    sv = jnp.dot(h2, w_ref[...], preferred_element_type=jnp.float32)
    st = lax.dot_general(w_ref[...], h2, (((0,), (1,)), ((), ())),
                         preferred_element_type=jnp.float32)
    out_ref[...] = sv + st + lb_ref[0, 0]


def _fin(A2, xp2, b2_row, lin_W, lin_b_2d):
    o2 = xp2.shape[1] // H
    return pl.pallas_call(
        functools.partial(_fin_body, o2),
        out_shape=jax.ShapeDtypeStruct((N, N), jnp.float32),
    )(A2, xp2, b2_row, lin_W, lin_b_2d)


def _abuild_body(src_hbm, dst_hbm, asn_hbm, adn_hbm, out_hbm,
                 src_v, dst_v, asn_v, adn_v, vals_v, idx_v, zero_v,
                 a_sh, sem_a, sem_z):
    c = lax.axis_index("c")
    s = lax.axis_index("s")

    ebase = s * EPW
    stage = [
        pltpu.async_copy(src_hbm.at[pl.ds(ebase, EPW)], src_v, sem_a),
        pltpu.async_copy(dst_hbm.at[pl.ds(ebase, EPW)], dst_v, sem_a),
        pltpu.async_copy(asn_hbm, asn_v, sem_a),
        pltpu.async_copy(adn_hbm, adn_v, sem_a),
    ]

    for i in range(ZBUF // 16):
        zero_v[pl.ds(i * 16, 16)] = jnp.zeros((16,), jnp.float32)
    abase = s * SLICE
    zcopies = [
        pltpu.async_copy(zero_v, a_sh.at[pl.ds(abase + k * ZBUF, ZBUF)], sem_z)
        for k in range(SLICE // ZBUF)
    ]

    for cp in stage:
        cp.wait()

    for r in range(NCH):
        for q in range(CHUNK // 16):
            off = r * CHUNK + q * 16
            s16 = src_v[pl.ds(off, 16)]
            d16 = dst_v[pl.ds(off, 16)]
            av = plsc.load_gather(asn_v, [s16 * H + c])
            bv = plsc.load_gather(adn_v, [d16 * H + c])
            e = av + bv
            e = jnp.where(e >= 0.0, e, e * 0.2)
            vals_v[r, pl.ds(q * 16, 16)] = jnp.exp(e)
            idx_v[r, pl.ds(q * 16, 16)] = d16 * N + s16

    for cp in zcopies:
        cp.wait()
    plsc.subcore_barrier()

    adds = [
        pltpu.async_copy(vals_v.at[r], a_sh.at[idx_v.at[r]], sem_a, add=True)
        for r in range(NCH)
    ]
    for cp in adds:
        cp.wait()

    plsc.subcore_barrier()
    pltpu.async_copy(a_sh.at[pl.ds(abase, SLICE)], out_hbm.at[c, s], sem_a).wait()


def _abuild(src, dst, asn_flat, adn_flat):
    mesh = plsc.VectorSubcoreMesh(core_axis_name="c", subcore_axis_name="s")
    k = pl.kernel(
        _abuild_body,
        out_type=jax.ShapeDtypeStruct((H, NS, SLICE), jnp.float32),
        mesh=mesh,
        compiler_params=pltpu.CompilerParams(needs_layout_passes=False),
        scratch_types=[
            pltpu.VMEM((EPW,), jnp.int32),
            pltpu.VMEM((EPW,), jnp.int32),
            pltpu.VMEM((N * H,), jnp.float32),
            pltpu.VMEM((N * H,), jnp.float32),
            pltpu.VMEM((NCH, CHUNK), jnp.float32),
            pltpu.VMEM((NCH, CHUNK), jnp.int32),
            pltpu.VMEM((ZBUF,), jnp.float32),
            pltpu.VMEM_SHARED((N * N,), jnp.float32),
            pltpu.SemaphoreType.DMA,
            pltpu.SemaphoreType.DMA,
        ],
    )
    return k(src, dst, asn_flat, adn_flat)


@jax.jit
def kernel(x_node, edge_index_node_to_node, W1, a_src1, a_dst1, b1,
           W2, a_src2, a_dst2, b2, lin_W, lin_b):
    src = edge_index_node_to_node[0].astype(jnp.int32)
    dst = edge_index_node_to_node[1].astype(jnp.int32)

    xp1, asn1, adn1 = _pre(x_node, W1, a_src1.reshape(1, -1), a_dst1.reshape(1, -1))
    A1 = _abuild(src, dst, asn1.reshape(-1), adn1.reshape(-1)).reshape(H, N, N)
    xp2, asn2, adn2 = _mid(A1, xp1, b1.reshape(1, -1), W2,
                           a_src2.reshape(1, -1), a_dst2.reshape(1, -1))
    A2 = _abuild(src, dst, asn2.reshape(-1), adn2.reshape(-1)).reshape(H, N, N)
    out = _fin(A2, xp2, b2.reshape(1, -1), lin_W, lin_b.reshape(1, 1))
    return out.reshape(N * N, 1)
